# Initial kernel scaffold; baseline (speedup 1.0000x reference)
import jax
import jax.numpy as jnp
from jax.experimental import pallas as pl


def kernel(categorical_inputs, numeric_inputs, tables, W1, b1, W2, b2, W3, b3, W4, b4):
    raise NotImplementedError("write your pallas kernel here")


# speedup vs baseline: 7.9563x; 7.9563x over previous
import functools

import jax
import jax.numpy as jnp
from jax import lax
from jax.experimental import pallas as pl
from jax.experimental.pallas import tpu as pltpu
from jax.experimental.pallas import tpu_sc as plsc

_BLK = 128
_K = 8


def _make_gather(NB, D):
    info = plsc.get_sparse_core_info()
    NC, NS = info.num_cores, info.num_subcores
    NW = NC * NS
    nb_w = NB // NW
    n_grp = nb_w // _K
    mesh = plsc.VectorSubcoreMesh(core_axis_name="c", subcore_axis_name="s")

    @functools.partial(
        pl.kernel,
        mesh=mesh,
        out_type=jax.ShapeDtypeStruct((NB, _BLK, D), jnp.float32),
        scratch_types=[
            pltpu.VMEM((_K, _BLK), jnp.int32),
            pltpu.VMEM((_K, _BLK, D), jnp.float32),
            pltpu.SemaphoreType.DMA,
        ],
        compiler_params=pltpu.CompilerParams(use_tc_tiling_on_sc=False),
    )
    def gather(tab_hbm, idx_hbm, out_hbm, idx_v, rows_v, sem):
        wid = lax.axis_index("s") * NC + lax.axis_index("c")
        base = wid * nb_w

        def group(g, carry):
            blk0 = base + g * _K
            pltpu.sync_copy(idx_hbm.at[pl.ds(blk0, _K)], idx_v)
            copies = [
                pltpu.async_copy(tab_hbm.at[idx_v.at[j]], rows_v.at[j], sem)
                for j in range(_K)
            ]
            for cp in copies:
                cp.wait()
            pltpu.sync_copy(rows_v, out_hbm.at[pl.ds(blk0, _K)])
            return carry

        lax.fori_loop(0, n_grp, group, 0)

    return gather


def _mlp(emb, num, W1a, W1b, b1, W2, b2, W3, b3, W4, b4, BM):
    Bt, E = emb.shape

    def body(emb_ref, num_ref, w1a_ref, w1b_ref, b1_ref, w2_ref, b2_ref,
             w3_ref, b3_ref, w4_ref, b4_ref, out_ref):
        h = jnp.dot(emb_ref[...], w1a_ref[...], preferred_element_type=jnp.float32)
        h = h + jnp.dot(num_ref[...], w1b_ref[...], preferred_element_type=jnp.float32)
        h = jnp.maximum(h + b1_ref[...], 0.0)
        h = jnp.maximum(jnp.dot(h, w2_ref[...], preferred_element_type=jnp.float32) + b2_ref[...], 0.0)
        h = jnp.maximum(jnp.dot(h, w3_ref[...], preferred_element_type=jnp.float32) + b3_ref[...], 0.0)
        out_ref[...] = jnp.dot(h, w4_ref[...], preferred_element_type=jnp.float32) + b4_ref[...]

    def full(a):
        nd = a.ndim
        return pl.BlockSpec(a.shape, lambda i, _nd=nd: (0,) * _nd)

    return pl.pallas_call(
        body,
        grid=(Bt // BM,),
        in_specs=[
            pl.BlockSpec((BM, E), lambda i: (i, 0)),
            pl.BlockSpec((BM, num.shape[1]), lambda i: (i, 0)),
            full(W1a), full(W1b), full(b1),
            full(W2), full(b2), full(W3), full(b3), full(W4), full(b4),
        ],
        out_specs=pl.BlockSpec((BM, 1), lambda i: (i, 0)),
        out_shape=jax.ShapeDtypeStruct((Bt, 1), jnp.float32),
    )(emb, num, W1a, W1b, b1, W2, b2, W3, b3, W4, b4)


def kernel(categorical_inputs, numeric_inputs, tables, W1, b1, W2, b2, W3, b3, W4, b4):
    B, F = categorical_inputs.shape
    _, V, D = tables.shape
    tab_flat = tables.reshape(F * V, D)
    flat_idx = (categorical_inputs + jnp.arange(F, dtype=jnp.int32) * V).reshape(-1)
    NB = (B * F) // _BLK
    idx_blocked = flat_idx.reshape(NB, _BLK)
    emb = _make_gather(NB, D)(tab_flat, idx_blocked)
    emb = emb.reshape(B, F * D)
    out = _mlp(
        emb, numeric_inputs,
        W1[: F * D], W1[F * D:], b1.reshape(1, -1),
        W2, b2.reshape(1, -1), W3, b3.reshape(1, -1), W4, b4.reshape(1, -1),
        BM=1024,
    )
    return out.reshape(B)
